# baseline (speedup 1.0000x reference)
import jax
import jax.numpy as jnp
from jax import lax
from jax.experimental import pallas as pl
from jax.experimental.pallas import tpu as pltpu
from jax.experimental.pallas import tpu_sc as plsc

_N_ENT = 40943
_EMB = 300
_EMBP = 320
_ROWS = 40960
_NPAN = 5
_PC = 64
_VOCAB = 30522
_K = 3
_DOUT = 768
_NTOK = 1024 * 50
_NC = 2
_NS = 16
_TPT = _NTOK // _NS
_C = 128
_NCHUNK = _TPT // _C
_STRIPE = _ROWS // _NS


def _sc_body(ids_hbm, m0_hbm, m1_hbm, m2_hbm, invtab_hbm,
             emb_hbm, pooled_hbm, inv_hbm,
             idsbuf, idx0, idx1, idx2, invbuf, r0, r1, r2, outb,
             panel, sem_map, sem_g):
    cid = lax.axis_index("c")
    sid = lax.axis_index("s")
    tok0 = sid * _TPT

    def idx_body(a, carry):
        base = tok0 + a * _C
        pltpu.sync_copy(ids_hbm.at[pl.ds(base, _C)], idsbuf)
        d0 = pltpu.async_copy(m0_hbm.at[idsbuf], idx0.at[pl.ds(a * _C, _C)], sem_map)
        d1 = pltpu.async_copy(m1_hbm.at[idsbuf], idx1.at[pl.ds(a * _C, _C)], sem_map)
        d2 = pltpu.async_copy(m2_hbm.at[idsbuf], idx2.at[pl.ds(a * _C, _C)], sem_map)
        d3 = pltpu.async_copy(invtab_hbm.at[idsbuf], invbuf.at[pl.ds(a * _C, _C)], sem_map)
        d0.wait()
        d1.wait()
        d2.wait()
        d3.wait()
        return carry

    lax.fori_loop(0, _NCHUNK, idx_body, 0)

    @pl.when(cid == 0)
    def _():
        pltpu.sync_copy(invbuf, inv_hbm.at[pl.ds(tok0, _TPT)])

    for p_local in range(3):
        if p_local < 2:
            p = cid * 2 + p_local
            c_lo = 0
            c_hi = _NCHUNK
        else:
            p = 4
            c_lo = cid * 13
            c_hi = 13 + cid * (_NCHUNK - 13)
        pltpu.sync_copy(emb_hbm.at[p, pl.ds(sid * _STRIPE, _STRIPE)],
                        panel.at[pl.ds(sid * _STRIPE, _STRIPE)])
        plsc.subcore_barrier()

        def chunk_body(a, carry):
            s = pl.ds(a * _C, _C)
            g0 = pltpu.async_copy(panel.at[idx0.at[s]], r0, sem_g)
            g1 = pltpu.async_copy(panel.at[idx1.at[s]], r1, sem_g)
            g2 = pltpu.async_copy(panel.at[idx2.at[s]], r2, sem_g)
            g0.wait()
            g1.wait()
            g2.wait()

            def tok_body(t, c2):
                outb[t, :] = r0[t, :] + r1[t, :] + r2[t, :]
                return c2

            lax.fori_loop(0, _C, tok_body, 0)
            pltpu.sync_copy(outb, pooled_hbm.at[p, pl.ds(tok0 + a * _C, _C)])
            return carry

        lax.fori_loop(c_lo, c_hi, chunk_body, 0)
        plsc.subcore_barrier()


def _sc_pool(ids_flat, m0, m1, m2, invtab, emb_panels):
    mesh = plsc.VectorSubcoreMesh(core_axis_name="c", subcore_axis_name="s",
                                  num_cores=_NC, num_subcores=_NS)
    f = pl.kernel(
        _sc_body,
        out_type=(jax.ShapeDtypeStruct((_NPAN, _NTOK, _PC), jnp.bfloat16),
                  jax.ShapeDtypeStruct((_NTOK,), jnp.float32)),
        mesh=mesh,
        compiler_params=pltpu.CompilerParams(use_tc_tiling_on_sc=False),
        scratch_types=[
            pltpu.VMEM((_C,), jnp.int32),
            pltpu.VMEM((_TPT,), jnp.int32),
            pltpu.VMEM((_TPT,), jnp.int32),
            pltpu.VMEM((_TPT,), jnp.int32),
            pltpu.VMEM((_TPT,), jnp.float32),
            pltpu.VMEM((_C, _PC), jnp.bfloat16),
            pltpu.VMEM((_C, _PC), jnp.bfloat16),
            pltpu.VMEM((_C, _PC), jnp.bfloat16),
            pltpu.VMEM((_C, _PC), jnp.bfloat16),
            pltpu.VMEM_SHARED((_ROWS, _PC), jnp.bfloat16),
            pltpu.SemaphoreType.DMA,
            pltpu.SemaphoreType.DMA,
        ],
    )
    return f(ids_flat, m0, m1, m2, invtab, emb_panels)


_RB = 256


def _mm_body(pooled_ref, inv_ref, w_ref, b_ref, x_ref, o_ref):
    xp = pooled_ref[...]
    w = jnp.concatenate([xp[p] for p in range(_NPAN)], axis=-1)
    acc = jnp.dot(w, w_ref[...], preferred_element_type=jnp.float32)
    o_ref[...] = acc * inv_ref[...] + b_ref[...] + x_ref[...]


def _project(pooled, inv, W_pad, b, x_flat):
    grid = (_NTOK // _RB,)
    return pl.pallas_call(
        _mm_body,
        grid=grid,
        in_specs=[
            pl.BlockSpec((_NPAN, _RB, _PC), lambda i: (0, i, 0)),
            pl.BlockSpec((_RB, 1), lambda i: (i, 0)),
            pl.BlockSpec((_EMBP, _DOUT), lambda i: (0, 0)),
            pl.BlockSpec((1, _DOUT), lambda i: (0, 0)),
            pl.BlockSpec((_RB, _DOUT), lambda i: (i, 0)),
        ],
        out_specs=pl.BlockSpec((_RB, _DOUT), lambda i: (i, 0)),
        out_shape=jax.ShapeDtypeStruct((_NTOK, _DOUT), jnp.float32),
    )(pooled, inv, W_pad, b, x_flat)


def kernel(input_ids, input_tensors, emb_table, map_ids, map_counts, W, b):
    B, L = input_ids.shape
    ids_flat = input_ids.reshape(-1).astype(jnp.int32)
    masked_ids = jnp.where(jnp.arange(_K, dtype=jnp.int32)[None, :]
                           < map_counts[:, None], map_ids, _N_ENT)
    inv_cnt = 1.0 / jnp.maximum(map_counts, 1).astype(jnp.float32)
    m0 = masked_ids[:, 0]
    m1 = masked_ids[:, 1]
    m2 = masked_ids[:, 2]
    emb_pad = jnp.zeros((_ROWS, _EMBP), jnp.bfloat16)
    emb_pad = emb_pad.at[:_N_ENT, :_EMB].set(emb_table.astype(jnp.bfloat16))
    emb_panels = emb_pad.reshape(_ROWS, _NPAN, _PC).transpose(1, 0, 2)

    pooled_panels, inv = _sc_pool(ids_flat, m0, m1, m2, inv_cnt, emb_panels)

    W_pad = jnp.zeros((_EMBP, _DOUT), jnp.bfloat16).at[:_EMB, :].set(
        W.astype(jnp.bfloat16))
    out = _project(pooled_panels, inv.reshape(-1, 1), W_pad, b.reshape(1, -1),
                   input_tensors.reshape(-1, _DOUT))
    return out.reshape(B, L, _DOUT)


# speedup vs baseline: 1.0595x; 1.0595x over previous
import jax
import jax.numpy as jnp
from jax import lax
from jax.experimental import pallas as pl
from jax.experimental.pallas import tpu as pltpu
from jax.experimental.pallas import tpu_sc as plsc

_N_ENT = 40943
_EMB = 300
_EMBP = 320
_ROWS = 40960
_NPAN = 5
_PC = 64
_VOCAB = 30522
_K = 3
_DOUT = 768
_NTOK = 1024 * 50
_NC = 2
_NS = 16
_TPT = _NTOK // _NS
_C = 128
_NCHUNK = _TPT // _C
_STRIPE = _ROWS // _NS


def _sc_body(ids_hbm, m0_hbm, m1_hbm, m2_hbm, invtab_hbm,
             emb_hbm, pooled_hbm, inv_hbm,
             idsbuf, idx0, idx1, idx2, invbuf, r0, r1, r2, outb,
             panel, sem_map, sem_g):
    cid = lax.axis_index("c")
    sid = lax.axis_index("s")
    tok0 = sid * _TPT

    def idx_body(a, carry):
        base = tok0 + a * _C
        pltpu.sync_copy(ids_hbm.at[pl.ds(base, _C)], idsbuf)
        d0 = pltpu.async_copy(m0_hbm.at[idsbuf], idx0.at[pl.ds(a * _C, _C)], sem_map)
        d1 = pltpu.async_copy(m1_hbm.at[idsbuf], idx1.at[pl.ds(a * _C, _C)], sem_map)
        d2 = pltpu.async_copy(m2_hbm.at[idsbuf], idx2.at[pl.ds(a * _C, _C)], sem_map)
        d3 = pltpu.async_copy(invtab_hbm.at[idsbuf], invbuf.at[pl.ds(a * _C, _C)], sem_map)
        d0.wait()
        d1.wait()
        d2.wait()
        d3.wait()
        return carry

    lax.fori_loop(0, _NCHUNK, idx_body, 0)

    @pl.when(cid == 0)
    def _():
        pltpu.sync_copy(invbuf, inv_hbm.at[pl.ds(tok0, _TPT)])

    for p_local in range(3):
        if p_local < 2:
            p = cid * 2 + p_local
            c_lo = 0
            c_hi = _NCHUNK
        else:
            p = 4
            c_lo = cid * 13
            c_hi = 13 + cid * (_NCHUNK - 13)
        pltpu.sync_copy(emb_hbm.at[p, pl.ds(sid * _STRIPE, _STRIPE)],
                        panel.at[pl.ds(sid * _STRIPE, _STRIPE)])
        plsc.subcore_barrier()

        def chunk_body(a, carry):
            s = pl.ds(a * _C, _C)
            g0 = pltpu.async_copy(panel.at[idx0.at[s]], r0, sem_g)
            g1 = pltpu.async_copy(panel.at[idx1.at[s]], r1, sem_g)
            g2 = pltpu.async_copy(panel.at[idx2.at[s]], r2, sem_g)
            g0.wait()
            g1.wait()
            g2.wait()

            def tok_body(t, c2):
                outb[t, :] = r0[t, :] + r1[t, :] + r2[t, :]
                return c2

            lax.fori_loop(0, _C, tok_body, 0)
            pltpu.sync_copy(outb, pooled_hbm.at[p, pl.ds(tok0 + a * _C, _C)])
            return carry

        lax.fori_loop(c_lo, c_hi, chunk_body, 0)
        plsc.subcore_barrier()


def _sc_pool(ids_flat, m0, m1, m2, invtab, emb_panels):
    mesh = plsc.VectorSubcoreMesh(core_axis_name="c", subcore_axis_name="s",
                                  num_cores=_NC, num_subcores=_NS)
    f = pl.kernel(
        _sc_body,
        out_type=(jax.ShapeDtypeStruct((_NPAN, _NTOK, _PC), jnp.bfloat16),
                  jax.ShapeDtypeStruct((_NTOK,), jnp.float32)),
        mesh=mesh,
        compiler_params=pltpu.CompilerParams(use_tc_tiling_on_sc=False),
        scratch_types=[
            pltpu.VMEM((_C,), jnp.int32),
            pltpu.VMEM((_TPT,), jnp.int32),
            pltpu.VMEM((_TPT,), jnp.int32),
            pltpu.VMEM((_TPT,), jnp.int32),
            pltpu.VMEM((_TPT,), jnp.float32),
            pltpu.VMEM((_C, _PC), jnp.bfloat16),
            pltpu.VMEM((_C, _PC), jnp.bfloat16),
            pltpu.VMEM((_C, _PC), jnp.bfloat16),
            pltpu.VMEM((_C, _PC), jnp.bfloat16),
            pltpu.VMEM_SHARED((_ROWS, _PC), jnp.bfloat16),
            pltpu.SemaphoreType.DMA,
            pltpu.SemaphoreType.DMA,
        ],
    )
    return f(ids_flat, m0, m1, m2, invtab, emb_panels)


_RB = 512


def _mm_body(pooled_ref, inv_ref, w_ref, b_ref, x_ref, o_ref):
    xp = pooled_ref[...]
    w = jnp.concatenate([xp[p] for p in range(_NPAN)], axis=-1)
    acc = jnp.dot(w, w_ref[...], preferred_element_type=jnp.float32)
    o_ref[...] = acc * inv_ref[...] + b_ref[...] + x_ref[...]


def _project(pooled, inv, W_pad, b, x_flat):
    grid = (_NTOK // _RB,)
    return pl.pallas_call(
        _mm_body,
        grid=grid,
        in_specs=[
            pl.BlockSpec((_NPAN, _RB, _PC), lambda i: (0, i, 0)),
            pl.BlockSpec((_RB, 1), lambda i: (i, 0)),
            pl.BlockSpec((_EMBP, _DOUT), lambda i: (0, 0)),
            pl.BlockSpec((1, _DOUT), lambda i: (0, 0)),
            pl.BlockSpec((_RB, _DOUT), lambda i: (i, 0)),
        ],
        out_specs=pl.BlockSpec((_RB, _DOUT), lambda i: (i, 0)),
        out_shape=jax.ShapeDtypeStruct((_NTOK, _DOUT), jnp.float32),
    )(pooled, inv, W_pad, b, x_flat)


def kernel(input_ids, input_tensors, emb_table, map_ids, map_counts, W, b):
    B, L = input_ids.shape
    ids_flat = input_ids.reshape(-1).astype(jnp.int32)
    masked_ids = jnp.where(jnp.arange(_K, dtype=jnp.int32)[None, :]
                           < map_counts[:, None], map_ids, _N_ENT)
    inv_cnt = 1.0 / jnp.maximum(map_counts, 1).astype(jnp.float32)
    m0 = masked_ids[:, 0]
    m1 = masked_ids[:, 1]
    m2 = masked_ids[:, 2]
    emb_pad = jnp.zeros((_ROWS, _EMBP), jnp.bfloat16)
    emb_pad = emb_pad.at[:_N_ENT, :_EMB].set(emb_table.astype(jnp.bfloat16))
    emb_panels = emb_pad.reshape(_ROWS, _NPAN, _PC).transpose(1, 0, 2)

    pooled_panels, inv = _sc_pool(ids_flat, m0, m1, m2, inv_cnt, emb_panels)

    W_pad = jnp.zeros((_EMBP, _DOUT), jnp.bfloat16).at[:_EMB, :].set(
        W.astype(jnp.bfloat16))
    out = _project(pooled_panels, inv.reshape(-1, 1), W_pad, b.reshape(1, -1),
                   input_tensors.reshape(-1, _DOUT))
    return out.reshape(B, L, _DOUT)


# speedup vs baseline: 1.0835x; 1.0227x over previous
import jax
import jax.numpy as jnp
from jax import lax
from jax.experimental import pallas as pl
from jax.experimental.pallas import tpu as pltpu
from jax.experimental.pallas import tpu_sc as plsc

_N_ENT = 40943
_EMB = 300
_EMBP = 320
_ROWS = 40960
_NPAN = 5
_PC = 64
_VOCAB = 30522
_K = 3
_DOUT = 768
_NTOK = 1024 * 50
_NC = 2
_NS = 16
_TPT = _NTOK // _NS
_C = 128
_NCHUNK = _TPT // _C
_STRIPE = _ROWS // _NS


def _sc_body(ids_hbm, m0_hbm, m1_hbm, m2_hbm, invtab_hbm,
             emb_hbm, pooled_hbm, inv_hbm,
             idsbuf, idx0, idx1, idx2, invbuf, r0, r1, r2, outb,
             panel, sem_map, sem_g):
    cid = lax.axis_index("c")
    sid = lax.axis_index("s")
    tok0 = sid * _TPT

    def idx_body(a, carry):
        base = tok0 + a * _C
        pltpu.sync_copy(ids_hbm.at[pl.ds(base, _C)], idsbuf)
        d0 = pltpu.async_copy(m0_hbm.at[idsbuf], idx0.at[pl.ds(a * _C, _C)], sem_map)
        d1 = pltpu.async_copy(m1_hbm.at[idsbuf], idx1.at[pl.ds(a * _C, _C)], sem_map)
        d2 = pltpu.async_copy(m2_hbm.at[idsbuf], idx2.at[pl.ds(a * _C, _C)], sem_map)
        d3 = pltpu.async_copy(invtab_hbm.at[idsbuf], invbuf.at[pl.ds(a * _C, _C)], sem_map)
        d0.wait()
        d1.wait()
        d2.wait()
        d3.wait()
        return carry

    lax.fori_loop(0, _NCHUNK, idx_body, 0)

    @pl.when(cid == 0)
    def _():
        pltpu.sync_copy(invbuf, inv_hbm.at[pl.ds(tok0, _TPT)])

    for p_local in range(3):
        if p_local < 2:
            p = cid * 2 + p_local
            c_lo = 0
            c_hi = _NCHUNK
        else:
            p = 4
            c_lo = cid * 13
            c_hi = 13 + cid * (_NCHUNK - 13)
        pltpu.sync_copy(emb_hbm.at[p, pl.ds(sid * _STRIPE, _STRIPE)],
                        panel.at[pl.ds(sid * _STRIPE, _STRIPE)])
        plsc.subcore_barrier()

        def chunk_body(a, carry):
            s = pl.ds(a * _C, _C)
            g0 = pltpu.async_copy(panel.at[idx0.at[s]], r0, sem_g)
            g1 = pltpu.async_copy(panel.at[idx1.at[s]], r1, sem_g)
            g2 = pltpu.async_copy(panel.at[idx2.at[s]], r2, sem_g)
            g0.wait()
            g1.wait()
            g2.wait()

            def tok_body(t, c2):
                outb[t, :] = r0[t, :] + r1[t, :] + r2[t, :]
                return c2

            lax.fori_loop(0, _C, tok_body, 0)
            pltpu.sync_copy(outb, pooled_hbm.at[p, pl.ds(tok0 + a * _C, _C)])
            return carry

        lax.fori_loop(c_lo, c_hi, chunk_body, 0)
        plsc.subcore_barrier()


def _sc_pool(ids_flat, m0, m1, m2, invtab, emb_panels):
    mesh = plsc.VectorSubcoreMesh(core_axis_name="c", subcore_axis_name="s",
                                  num_cores=_NC, num_subcores=_NS)
    f = pl.kernel(
        _sc_body,
        out_type=(jax.ShapeDtypeStruct((_NPAN, _NTOK, _PC), jnp.bfloat16),
                  jax.ShapeDtypeStruct((_NTOK,), jnp.float32)),
        mesh=mesh,
        compiler_params=pltpu.CompilerParams(use_tc_tiling_on_sc=False),
        scratch_types=[
            pltpu.VMEM((_C,), jnp.int32),
            pltpu.VMEM((_TPT,), jnp.int32),
            pltpu.VMEM((_TPT,), jnp.int32),
            pltpu.VMEM((_TPT,), jnp.int32),
            pltpu.VMEM((_TPT,), jnp.float32),
            pltpu.VMEM((_C, _PC), jnp.bfloat16),
            pltpu.VMEM((_C, _PC), jnp.bfloat16),
            pltpu.VMEM((_C, _PC), jnp.bfloat16),
            pltpu.VMEM((_C, _PC), jnp.bfloat16),
            pltpu.VMEM_SHARED((_ROWS, _PC), jnp.bfloat16),
            pltpu.SemaphoreType.DMA,
            pltpu.SemaphoreType.DMA,
        ],
    )
    return f(ids_flat, m0, m1, m2, invtab, emb_panels)


_RB = 1024


def _mm_body(pooled_ref, inv_ref, w_ref, b_ref, x_ref, o_ref):
    xp = pooled_ref[...]
    w = jnp.concatenate([xp[p] for p in range(_NPAN)], axis=-1)
    acc = jnp.dot(w, w_ref[...], preferred_element_type=jnp.float32)
    o_ref[...] = acc * inv_ref[...] + b_ref[...] + x_ref[...]


def _project(pooled, inv, W_pad, b, x_flat):
    grid = (_NTOK // _RB,)
    return pl.pallas_call(
        _mm_body,
        grid=grid,
        in_specs=[
            pl.BlockSpec((_NPAN, _RB, _PC), lambda i: (0, i, 0)),
            pl.BlockSpec((_RB, 1), lambda i: (i, 0)),
            pl.BlockSpec((_EMBP, _DOUT), lambda i: (0, 0)),
            pl.BlockSpec((1, _DOUT), lambda i: (0, 0)),
            pl.BlockSpec((_RB, _DOUT), lambda i: (i, 0)),
        ],
        out_specs=pl.BlockSpec((_RB, _DOUT), lambda i: (i, 0)),
        out_shape=jax.ShapeDtypeStruct((_NTOK, _DOUT), jnp.float32),
    )(pooled, inv, W_pad, b, x_flat)


def kernel(input_ids, input_tensors, emb_table, map_ids, map_counts, W, b):
    B, L = input_ids.shape
    ids_flat = input_ids.reshape(-1).astype(jnp.int32)
    masked_ids = jnp.where(jnp.arange(_K, dtype=jnp.int32)[None, :]
                           < map_counts[:, None], map_ids, _N_ENT)
    inv_cnt = 1.0 / jnp.maximum(map_counts, 1).astype(jnp.float32)
    m0 = masked_ids[:, 0]
    m1 = masked_ids[:, 1]
    m2 = masked_ids[:, 2]
    emb_pad = jnp.zeros((_ROWS, _EMBP), jnp.bfloat16)
    emb_pad = emb_pad.at[:_N_ENT, :_EMB].set(emb_table.astype(jnp.bfloat16))
    emb_panels = emb_pad.reshape(_ROWS, _NPAN, _PC).transpose(1, 0, 2)

    pooled_panels, inv = _sc_pool(ids_flat, m0, m1, m2, inv_cnt, emb_panels)

    W_pad = jnp.zeros((_EMBP, _DOUT), jnp.bfloat16).at[:_EMB, :].set(
        W.astype(jnp.bfloat16))
    out = _project(pooled_panels, inv.reshape(-1, 1), W_pad, b.reshape(1, -1),
                   input_tensors.reshape(-1, _DOUT))
    return out.reshape(B, L, _DOUT)


# speedup vs baseline: 1.0888x; 1.0048x over previous
import jax
import jax.numpy as jnp
from jax import lax
from jax.experimental import pallas as pl
from jax.experimental.pallas import tpu as pltpu
from jax.experimental.pallas import tpu_sc as plsc

_N_ENT = 40943
_EMB = 300
_EMBP = 320
_ROWS = 40960
_NPAN = 5
_PC = 64
_VOCAB = 30522
_K = 3
_DOUT = 768
_NTOK = 1024 * 50
_NC = 2
_NS = 16
_TPT = _NTOK // _NS
_C = 128
_NCHUNK = _TPT // _C
_STRIPE = _ROWS // _NS


def _sc_body(ids_hbm, m0_hbm, m1_hbm, m2_hbm, invtab_hbm,
             emb_hbm, pooled_hbm, inv_hbm,
             idsbuf, idx0, idx1, idx2, invbuf, r0, r1, r2, outb,
             panel, sem_map, sem_g):
    cid = lax.axis_index("c")
    sid = lax.axis_index("s")
    tok0 = sid * _TPT

    def idx_body(a, carry):
        base = tok0 + a * _C
        pltpu.sync_copy(ids_hbm.at[pl.ds(base, _C)], idsbuf)
        d0 = pltpu.async_copy(m0_hbm.at[idsbuf], idx0.at[pl.ds(a * _C, _C)], sem_map)
        d1 = pltpu.async_copy(m1_hbm.at[idsbuf], idx1.at[pl.ds(a * _C, _C)], sem_map)
        d2 = pltpu.async_copy(m2_hbm.at[idsbuf], idx2.at[pl.ds(a * _C, _C)], sem_map)
        d3 = pltpu.async_copy(invtab_hbm.at[idsbuf], invbuf.at[pl.ds(a * _C, _C)], sem_map)
        d0.wait()
        d1.wait()
        d2.wait()
        d3.wait()
        return carry

    lax.fori_loop(0, _NCHUNK, idx_body, 0)

    @pl.when(cid == 0)
    def _():
        pltpu.sync_copy(invbuf, inv_hbm.at[pl.ds(tok0, _TPT)])

    for p_local in range(3):
        if p_local < 2:
            p = cid * 2 + p_local
            c_lo = 0
            c_hi = _NCHUNK
        else:
            p = 4
            c_lo = cid * 13
            c_hi = 13 + cid * (_NCHUNK - 13)
        pltpu.sync_copy(emb_hbm.at[p, pl.ds(sid * _STRIPE, _STRIPE)],
                        panel.at[pl.ds(sid * _STRIPE, _STRIPE)])
        plsc.subcore_barrier()

        def chunk_body(a, carry):
            s = pl.ds(a * _C, _C)
            g0 = pltpu.async_copy(panel.at[idx0.at[s]], r0, sem_g)
            g1 = pltpu.async_copy(panel.at[idx1.at[s]], r1, sem_g)
            g2 = pltpu.async_copy(panel.at[idx2.at[s]], r2, sem_g)
            g0.wait()
            g1.wait()
            g2.wait()

            def tok_body(t, c2):
                outb[t, :] = r0[t, :] + r1[t, :] + r2[t, :]
                return c2

            lax.fori_loop(0, _C, tok_body, 0)
            pltpu.sync_copy(outb, pooled_hbm.at[p, pl.ds(tok0 + a * _C, _C)])
            return carry

        lax.fori_loop(c_lo, c_hi, chunk_body, 0)
        plsc.subcore_barrier()


def _sc_pool(ids_flat, m0, m1, m2, invtab, emb_panels):
    mesh = plsc.VectorSubcoreMesh(core_axis_name="c", subcore_axis_name="s",
                                  num_cores=_NC, num_subcores=_NS)
    f = pl.kernel(
        _sc_body,
        out_type=(jax.ShapeDtypeStruct((_NPAN, _NTOK, _PC), jnp.bfloat16),
                  jax.ShapeDtypeStruct((_NTOK,), jnp.float32)),
        mesh=mesh,
        compiler_params=pltpu.CompilerParams(use_tc_tiling_on_sc=False),
        scratch_types=[
            pltpu.VMEM((_C,), jnp.int32),
            pltpu.VMEM((_TPT,), jnp.int32),
            pltpu.VMEM((_TPT,), jnp.int32),
            pltpu.VMEM((_TPT,), jnp.int32),
            pltpu.VMEM((_TPT,), jnp.float32),
            pltpu.VMEM((_C, _PC), jnp.bfloat16),
            pltpu.VMEM((_C, _PC), jnp.bfloat16),
            pltpu.VMEM((_C, _PC), jnp.bfloat16),
            pltpu.VMEM((_C, _PC), jnp.bfloat16),
            pltpu.VMEM_SHARED((_ROWS, _PC), jnp.bfloat16),
            pltpu.SemaphoreType.DMA,
            pltpu.SemaphoreType.DMA,
        ],
    )
    return f(ids_flat, m0, m1, m2, invtab, emb_panels)


_RB = 2048


def _mm_body(pooled_ref, inv_ref, w_ref, b_ref, x_ref, o_ref):
    xp = pooled_ref[...]
    w = jnp.concatenate([xp[p] for p in range(_NPAN)], axis=-1)
    acc = jnp.dot(w, w_ref[...], preferred_element_type=jnp.float32)
    o_ref[...] = acc * inv_ref[...] + b_ref[...] + x_ref[...]


def _project(pooled, inv, W_pad, b, x_flat):
    grid = (_NTOK // _RB,)
    return pl.pallas_call(
        _mm_body,
        grid=grid,
        in_specs=[
            pl.BlockSpec((_NPAN, _RB, _PC), lambda i: (0, i, 0)),
            pl.BlockSpec((_RB, 1), lambda i: (i, 0)),
            pl.BlockSpec((_EMBP, _DOUT), lambda i: (0, 0)),
            pl.BlockSpec((1, _DOUT), lambda i: (0, 0)),
            pl.BlockSpec((_RB, _DOUT), lambda i: (i, 0)),
        ],
        out_specs=pl.BlockSpec((_RB, _DOUT), lambda i: (i, 0)),
        out_shape=jax.ShapeDtypeStruct((_NTOK, _DOUT), jnp.float32),
    )(pooled, inv, W_pad, b, x_flat)


def kernel(input_ids, input_tensors, emb_table, map_ids, map_counts, W, b):
    B, L = input_ids.shape
    ids_flat = input_ids.reshape(-1).astype(jnp.int32)
    masked_ids = jnp.where(jnp.arange(_K, dtype=jnp.int32)[None, :]
                           < map_counts[:, None], map_ids, _N_ENT)
    inv_cnt = 1.0 / jnp.maximum(map_counts, 1).astype(jnp.float32)
    m0 = masked_ids[:, 0]
    m1 = masked_ids[:, 1]
    m2 = masked_ids[:, 2]
    emb_pad = jnp.zeros((_ROWS, _EMBP), jnp.bfloat16)
    emb_pad = emb_pad.at[:_N_ENT, :_EMB].set(emb_table.astype(jnp.bfloat16))
    emb_panels = emb_pad.reshape(_ROWS, _NPAN, _PC).transpose(1, 0, 2)

    pooled_panels, inv = _sc_pool(ids_flat, m0, m1, m2, inv_cnt, emb_panels)

    W_pad = jnp.zeros((_EMBP, _DOUT), jnp.bfloat16).at[:_EMB, :].set(
        W.astype(jnp.bfloat16))
    out = _project(pooled_panels, inv.reshape(-1, 1), W_pad, b.reshape(1, -1),
                   input_tensors.reshape(-1, _DOUT))
    return out.reshape(B, L, _DOUT)
